# baseline (speedup 1.0000x reference)
import functools

import jax
import jax.numpy as jnp
import numpy as np
from jax import lax
from jax.experimental import pallas as pl
from jax.experimental.pallas import tpu as pltpu
from jax.experimental.pallas import tpu_sc as plsc

_N = 10000
_E = 160000
_D = 256
_H = 512
_L = 128
_HC = 64
_NC = 2
_NS = 16
_NW = _NC * _NS
_CH = 128
_CPW = 40
_EW = _CH * _CPW
_EPAD = _EW * _NW
_NACC = 10240
_NACC64 = 10048
_RPT = _N // _NS
_BT = 1000
_NBUF = 4


@functools.cache
def _mesh():
    return plsc.VectorSubcoreMesh(core_axis_name="c", subcore_axis_name="s",
                                  num_cores=_NC, num_subcores=_NS)


@functools.cache
def _make_prop(nb):
    out_type = [jax.ShapeDtypeStruct((_NC, _N, _HC), jnp.float32) for _ in range(nb)]
    scratch = (
        [pltpu.VMEM((_CPW, _CH), jnp.int32),
         pltpu.VMEM((_CPW, _CH), jnp.int32)]
        + [pltpu.VMEM((_CH, _HC), jnp.float32) for _ in range(_NBUF)]
        + [pltpu.VMEM_SHARED((_N, _HC), jnp.float32),
           pltpu.VMEM_SHARED((_NACC64, _HC), jnp.float32)]
        + [pltpu.SemaphoreType.DMA for _ in range(2 * _NBUF)]
    )

    @functools.partial(pl.kernel, mesh=_mesh(), out_type=out_type,
                       scratch_types=scratch,
                       compiler_params=pltpu.CompilerParams(use_tc_tiling_on_sc=False))
    def prop(*refs):
        u = refs[:nb]
        src_hbm = refs[nb]
        dst_hbm = refs[nb + 1]
        outs = refs[nb + 2: 2 * nb + 2]
        rest = refs[2 * nb + 2:]
        src_v, dst_v = rest[0], rest[1]
        rows = rest[2:2 + _NBUF]
        tab = rest[2 + _NBUF]
        acc = rest[3 + _NBUF]
        gsem = rest[4 + _NBUF: 4 + 2 * _NBUF]
        ssem = rest[4 + 2 * _NBUF:]
        c = lax.axis_index("c")
        s = lax.axis_index("s")
        wid = s * _NC + c
        rr = pl.ds(s * _RPT, _RPT)
        pltpu.sync_copy(src_hbm.at[wid], src_v)
        pltpu.sync_copy(dst_hbm.at[wid], dst_v)

        def fire_gather(k, b):
            pltpu.async_copy(tab.at[src_v.at[k]], rows[b], gsem[b])

        def drain(j, sem):
            pltpu.make_async_copy(u[j].at[pl.ds(0, _CH)], rows[0], sem).wait()

        for j in range(nb):
            pltpu.sync_copy(u[j].at[rr], tab.at[rr])
            pltpu.sync_copy(u[j].at[rr], acc.at[rr])
            plsc.subcore_barrier()
            fire_gather(0, 0)
            fire_gather(1, 1)

            def step(k, b, j=j):
                drain(j, gsem[b])
                pltpu.async_copy(rows[b], acc.at[dst_v.at[k]], ssem[b],
                                 add=True)

            def prefetch(k2, b2, j=j, first=False):
                if not first:
                    drain(j, ssem[b2])
                fire_gather(k2, b2)

            step(0, 0)
            prefetch(2, 2, first=True)
            step(1, 1)
            prefetch(3, 3, first=True)

            def body(g, carry):
                for i in range(_NBUF):
                    k = 2 + g * _NBUF + i
                    step(k, (2 + i) % _NBUF)
                    prefetch(k + 2, i % _NBUF)
                return carry

            lax.fori_loop(0, (_CPW - 4) // _NBUF, body, 0)
            step(_CPW - 2, (_CPW - 2) % _NBUF)
            step(_CPW - 1, (_CPW - 1) % _NBUF)
            for b in range(_NBUF):
                drain(j, ssem[b])
            plsc.subcore_barrier()
            pltpu.sync_copy(acc.at[rr], outs[j].at[c, rr])

    return prop


@functools.cache
def _make_deg():
    @functools.partial(
        pl.kernel, mesh=_mesh(),
        out_type=jax.ShapeDtypeStruct((_NC, _N, 8), jnp.float32),
        scratch_types=[
            pltpu.VMEM((_CPW, _CH), jnp.int32),
            pltpu.VMEM((_CH, 8), jnp.float32),
            pltpu.VMEM_SHARED((_NACC, 8), jnp.float32),
            pltpu.SemaphoreType.DMA,
        ],
        compiler_params=pltpu.CompilerParams(use_tc_tiling_on_sc=False))
    def deg(dst_hbm, ones_hbm, out, dst_v, rows_v, acc, sem):
        c = lax.axis_index("c")
        s = lax.axis_index("s")
        wid = s * _NC + c
        pltpu.sync_copy(dst_hbm.at[wid], dst_v)
        pltpu.sync_copy(ones_hbm.at[pl.ds(0, _CH), pl.ds(0, 8)], rows_v)
        pltpu.sync_copy(ones_hbm.at[pl.ds(0, _RPT), pl.ds(0, 8)],
                        acc.at[pl.ds(s * _RPT, _RPT)])
        plsc.subcore_barrier()

        def fire(k, carry):
            pltpu.async_copy(rows_v, acc.at[dst_v.at[k]], sem, add=True)
            return carry

        def drn(k, carry):
            pltpu.make_async_copy(ones_hbm.at[pl.ds(0, _CH), pl.ds(0, 8)],
                                  rows_v, sem).wait()
            return carry

        lax.fori_loop(0, _CPW, fire, 0)
        lax.fori_loop(0, _CPW, drn, 0)
        plsc.subcore_barrier()
        pltpu.sync_copy(acc.at[pl.ds(s * _RPT, _RPT)],
                        out.at[c, pl.ds(s * _RPT, _RPT)])

    return deg


def _whole(shape):
    return pl.BlockSpec(shape, lambda i: tuple(0 for _ in shape))


def _rows(shape):
    nd = len(shape)
    return pl.BlockSpec((_BT,) + shape[1:], lambda i: (i,) + tuple(0 for _ in range(nd - 1)))


def _mid(shape):
    return pl.BlockSpec((shape[0], _BT) + shape[2:],
                        lambda i: (0, i) + tuple(0 for _ in range(len(shape) - 2)))


def _tc1_body(deg_ref, x_ref, dinv_ref, *u_refs):
    d = deg_ref[0, :, 0:1] + deg_ref[1, :, 0:1] - 1.0
    dinv = lax.rsqrt(d)
    dinv_ref[...] = dinv
    u = x_ref[...] * dinv
    for i, r in enumerate(u_refs):
        r[...] = u[:, i * _HC:(i + 1) * _HC]


def _tc1(deg2, x):
    nu = _D // _HC
    return pl.pallas_call(
        _tc1_body,
        grid=(_N // _BT,),
        in_specs=[_mid((_NC, _N, 8)), _rows((_N, _D))],
        out_specs=[_rows((_N, 1))] + [_rows((_N, _HC))] * nu,
        out_shape=[jax.ShapeDtypeStruct((_N, 1), jnp.float32)]
                  + [jax.ShapeDtypeStruct((_N, _HC), jnp.float32)] * nu,
    )(deg2, x)


def _combine(s_refs, u_refs, dv):
    return jnp.concatenate(
        [(s[0] + s[1] - u[...]) * dv for s, u in zip(s_refs, u_refs)], axis=1)


def _tc2_body(*refs):
    s_refs = refs[:4]
    u_refs = refs[4:8]
    dinv, w1, b1 = refs[8:11]
    outs = refs[11:]
    dv = dinv[...]
    t = _combine(s_refs, u_refs, dv)
    h = jnp.dot(t, w1[...], preferred_element_type=jnp.float32) + b1[...]
    h = jnp.maximum(h, 0.0) * dv
    for i, o in enumerate(outs):
        o[...] = h[:, i * _HC:(i + 1) * _HC]


def _tc2(s, u, dinv, w1, b1):
    return pl.pallas_call(
        _tc2_body,
        grid=(_N // _BT,),
        in_specs=[_mid((_NC, _N, _HC))] * 4
                 + [_rows((_N, _HC))] * 4
                 + [_rows((_N, 1)), _whole((_D, _H)), _whole((1, _H))],
        out_specs=[_rows((_N, _HC))] * 8,
        out_shape=[jax.ShapeDtypeStruct((_N, _HC), jnp.float32)] * 8,
    )(*s, *u, dinv, w1, b1)


def _tc3_body(*refs):
    s_refs = refs[:8]
    u_refs = refs[8:16]
    dinv, w2, b2, wc = refs[16:20]
    outs = refs[20:]
    dv = dinv[...]
    t = _combine(s_refs, u_refs, dv)
    h = jnp.dot(t, w2[...], preferred_element_type=jnp.float32) + b2[...]
    h = jnp.maximum(h, 0.0)
    cc = jnp.dot(h, wc[...], preferred_element_type=jnp.float32) * dv
    for i, o in enumerate(outs):
        o[...] = cc[:, i * _HC:(i + 1) * _HC]


def _tc3(s, u, dinv, w2, b2, wc):
    return pl.pallas_call(
        _tc3_body,
        grid=(_N // _BT,),
        in_specs=[_mid((_NC, _N, _HC))] * 8
                 + [_rows((_N, _HC))] * 8
                 + [_rows((_N, 1)), _whole((_H, _H)), _whole((1, _H)),
                    _whole((_H, 2 * _L))],
        out_specs=[_rows((_N, _HC))] * 4,
        out_shape=[jax.ShapeDtypeStruct((_N, _HC), jnp.float32)] * 4,
    )(*s, *u, dinv, w2, b2, wc)


def _tc4_body(*refs):
    s_refs = refs[:4]
    u_refs = refs[4:8]
    (dinv, bmu, blv, eps, wd1, bd1, wd2, bd2, wd3, bd3,
     mu_o, lv_o, z_o, rec_o, zacc) = refs[8:]
    i = pl.program_id(0)
    dv = dinv[...]
    t = _combine(s_refs, u_refs, dv)
    mu = t[:, :_L] + bmu[...]
    lv = t[:, _L:] + blv[...]
    std = jnp.exp(0.5 * lv)
    z = mu + eps[...] * std
    mu_o[...] = mu
    lv_o[...] = lv
    z_o[...] = z

    @pl.when(i == 0)
    def _():
        zacc[...] = jnp.zeros_like(zacc)

    zacc[...] += jnp.sum(z, axis=0, keepdims=True)

    @pl.when(i == pl.num_programs(0) - 1)
    def _():
        ge = zacc[...] * (1.0 / _N)
        d1 = jnp.dot(ge, wd1[...], preferred_element_type=jnp.float32) + bd1[...]
        d1 = jnp.maximum(d1, 0.0)
        d2 = jnp.dot(d1, wd2[...], preferred_element_type=jnp.float32) + bd2[...]
        d2 = jnp.maximum(d2, 0.0)
        o = jnp.dot(d2, wd3[...], preferred_element_type=jnp.float32) + bd3[...]
        rec_o[...] = 1.0 / (1.0 + jnp.exp(-o))


def _tc4(s, u, dinv, bmu, blv, eps, wd1, bd1, wd2, bd2, wd3, bd3):
    return pl.pallas_call(
        _tc4_body,
        grid=(_N // _BT,),
        in_specs=[_mid((_NC, _N, _HC))] * 4
                 + [_rows((_N, _HC))] * 4
                 + [_rows((_N, 1)), _whole((1, _L)), _whole((1, _L)),
                    _rows((_N, _L)),
                    _whole((_L, _H)), _whole((1, _H)),
                    _whole((_H, _H)), _whole((1, _H)),
                    _whole((_H, _D)), _whole((1, _D))],
        out_specs=[_rows((_N, _L)), _rows((_N, _L)), _rows((_N, _L)),
                   _whole((1, _D))],
        out_shape=[jax.ShapeDtypeStruct((_N, _L), jnp.float32),
                   jax.ShapeDtypeStruct((_N, _L), jnp.float32),
                   jax.ShapeDtypeStruct((_N, _L), jnp.float32),
                   jax.ShapeDtypeStruct((1, _D), jnp.float32)],
        scratch_shapes=[pltpu.VMEM((1, _L), jnp.float32)],
    )(*s, *u, dinv, bmu, blv, eps, wd1, bd1, wd2, bd2, wd3, bd3)


def kernel(x, edge_index, W1, b1, W2, b2, Wmu, bmu, Wlv, blv,
           Wd1, bd1, Wd2, bd2, Wd3, bd3):
    f32 = jnp.float32
    src = edge_index[0]
    dst = edge_index[1]
    npad = _EPAD - _E
    src_p = jnp.concatenate([src, jnp.zeros((npad,), jnp.int32)]).reshape(_NW, _CPW, _CH)
    dst_p = jnp.concatenate([dst, jnp.full((npad,), _N, jnp.int32)]).reshape(_NW, _CPW, _CH)

    ones = jnp.ones((_RPT, 8), f32)
    deg2 = _make_deg()(dst_p, ones)

    dinv, *u = _tc1(deg2, x)
    s = _make_prop(4)(*u, src_p, dst_p)
    v = _tc2(s, u, dinv, W1, b1.reshape(1, _H))
    t = _make_prop(8)(*v, src_p, dst_p)
    wc = jnp.concatenate([Wmu, Wlv], axis=1)
    w = _tc3(t, v, dinv, W2, b2.reshape(1, _H), wc)
    r = _make_prop(4)(*w, src_p, dst_p)
    eps = jax.random.normal(jax.random.key(42), (_N, _L), dtype=f32)
    mu, lv, z, rec = _tc4(r, w, dinv,
                          bmu.reshape(1, _L), blv.reshape(1, _L), eps,
                          Wd1, bd1.reshape(1, _H), Wd2, bd2.reshape(1, _H),
                          Wd3, bd3.reshape(1, _D))
    return rec, mu, lv, z


# speedup vs baseline: 1.3059x; 1.3059x over previous
import functools

import jax
import jax.numpy as jnp
import numpy as np
from jax import lax
from jax.experimental import pallas as pl
from jax.experimental.pallas import tpu as pltpu
from jax.experimental.pallas import tpu_sc as plsc

_N = 10000
_E = 160000
_D = 256
_H = 512
_L = 128
_CB = 128
_NC = 2
_NS = 16
_NW = _NC * _NS
_CH = 128
_CPW = 40
_CHD = 128
_CPWD = 40
_EW = _CH * _CPW
_EPAD = _EW * _NW
_NACC = 10240
_RPT = _N // _NS
_BT = 2000


@functools.cache
def _mesh():
    return plsc.VectorSubcoreMesh(core_axis_name="c", subcore_axis_name="s",
                                  num_cores=_NC, num_subcores=_NS)


_NBUF = 4
_HC = 64
_NACC64 = 10048


@functools.cache
def _make_prop(nb):
    out_type = [jax.ShapeDtypeStruct((_NC, _N, _CB), jnp.float32) for _ in range(nb)]
    scratch = (
        [pltpu.VMEM((_CPW, _CH), jnp.int32),
         pltpu.VMEM((_CPW, _CH), jnp.int32)]
        + [pltpu.VMEM((_CH, _HC), jnp.float32) for _ in range(_NBUF)]
        + [pltpu.VMEM_SHARED((_N, _HC), jnp.float32),
           pltpu.VMEM_SHARED((_NACC64, _HC), jnp.float32)]
        + [pltpu.SemaphoreType.DMA for _ in range(2 * _NBUF)]
    )

    @functools.partial(pl.kernel, mesh=_mesh(), out_type=out_type,
                       scratch_types=scratch,
                       compiler_params=pltpu.CompilerParams(use_tc_tiling_on_sc=False))
    def prop(*refs):
        u = refs[:nb]
        src_hbm = refs[nb]
        dst_hbm = refs[nb + 1]
        outs = refs[nb + 2: 2 * nb + 2]
        rest = refs[2 * nb + 2:]
        src_v, dst_v = rest[0], rest[1]
        rows = rest[2:2 + _NBUF]
        tab = rest[2 + _NBUF]
        acc = rest[3 + _NBUF]
        gsem = rest[4 + _NBUF: 4 + 2 * _NBUF]
        ssem = rest[4 + 2 * _NBUF:]
        c = lax.axis_index("c")
        s = lax.axis_index("s")
        wid = s * _NC + c
        rr = pl.ds(s * _RPT, _RPT)
        pltpu.sync_copy(src_hbm.at[wid], src_v)
        pltpu.sync_copy(dst_hbm.at[wid], dst_v)

        def fire_gather(k, b):
            pltpu.async_copy(tab.at[src_v.at[k]], rows[b], gsem[b])

        def drain(j, sem):
            pltpu.make_async_copy(u[j].at[pl.ds(0, _CH), pl.ds(0, _HC)],
                                  rows[0], sem).wait()

        for j in range(nb):
            for h in range(2):
                col = pl.ds(h * _HC, _HC)
                pltpu.sync_copy(u[j].at[rr, col], tab.at[rr])
                pltpu.sync_copy(u[j].at[rr, col], acc.at[rr])
                plsc.subcore_barrier()
                fire_gather(0, 0)
                fire_gather(1, 1)

                def step(k, b, j=j):
                    drain(j, gsem[b])
                    pltpu.async_copy(rows[b], acc.at[dst_v.at[k]], ssem[b],
                                     add=True)

                def prefetch(k2, b2, j=j, first=False):
                    if not first:
                        drain(j, ssem[b2])
                    fire_gather(k2, b2)

                step(0, 0)
                prefetch(2, 2, first=True)
                step(1, 1)
                prefetch(3, 3, first=True)

                def body(g, carry):
                    for i in range(_NBUF):
                        k = 2 + g * _NBUF + i
                        step(k, (2 + i) % _NBUF)
                        prefetch(k + 2, i % _NBUF)
                    return carry

                lax.fori_loop(0, (_CPW - 4) // _NBUF, body, 0)
                step(_CPW - 2, (_CPW - 2) % _NBUF)
                step(_CPW - 1, (_CPW - 1) % _NBUF)
                for b in range(_NBUF):
                    drain(j, ssem[b])
                plsc.subcore_barrier()
                pltpu.sync_copy(acc.at[rr],
                                outs[j].at[c, rr, col])

    return prop


@functools.cache
def _make_deg():
    @functools.partial(
        pl.kernel, mesh=_mesh(),
        out_type=jax.ShapeDtypeStruct((_NC, _N, 8), jnp.float32),
        scratch_types=[
            pltpu.VMEM((_CPWD, _CHD), jnp.int32),
            pltpu.VMEM((_CHD, 8), jnp.float32),
            pltpu.VMEM_SHARED((_NACC, 8), jnp.float32),
            pltpu.SemaphoreType.DMA,
        ],
        compiler_params=pltpu.CompilerParams(use_tc_tiling_on_sc=False))
    def deg(dst_hbm, ones_hbm, out, dst_v, rows_v, acc, sem):
        c = lax.axis_index("c")
        s = lax.axis_index("s")
        wid = s * _NC + c
        pltpu.sync_copy(dst_hbm.at[wid], dst_v)
        pltpu.sync_copy(ones_hbm.at[pl.ds(0, _CHD), pl.ds(0, 8)], rows_v)
        pltpu.sync_copy(ones_hbm.at[pl.ds(0, _RPT), pl.ds(0, 8)],
                        acc.at[pl.ds(s * _RPT, _RPT)])
        plsc.subcore_barrier()

        def fire(k, carry):
            pltpu.async_copy(rows_v, acc.at[dst_v.at[k]], sem, add=True)
            return carry

        def drn(k, carry):
            pltpu.make_async_copy(ones_hbm.at[pl.ds(0, _CHD), pl.ds(0, 8)],
                                  rows_v, sem).wait()
            return carry

        lax.fori_loop(0, _CPWD, fire, 0)
        lax.fori_loop(0, _CPWD, drn, 0)
        plsc.subcore_barrier()
        pltpu.sync_copy(acc.at[pl.ds(s * _RPT, _RPT)],
                        out.at[c, pl.ds(s * _RPT, _RPT)])

    return deg


def _whole(shape):
    return pl.BlockSpec(shape, lambda i: tuple(0 for _ in shape))


def _rows(shape):
    nd = len(shape)
    return pl.BlockSpec((_BT,) + shape[1:], lambda i: (i,) + tuple(0 for _ in range(nd - 1)))


def _mid(shape):
    return pl.BlockSpec((shape[0], _BT) + shape[2:],
                        lambda i: (0, i) + tuple(0 for _ in range(len(shape) - 2)))


def _tc1_body(deg_ref, x_ref, dinv_ref, u0_ref, u1_ref):
    d = deg_ref[0, :, 0:1] + deg_ref[1, :, 0:1] - 1.0
    dinv = lax.rsqrt(d)
    dinv_ref[...] = dinv
    u = x_ref[...] * dinv
    u0_ref[...] = u[:, :_CB]
    u1_ref[...] = u[:, _CB:]


def _tc1(deg2, x):
    return pl.pallas_call(
        _tc1_body,
        grid=(_N // _BT,),
        in_specs=[_mid((_NC, _N, 8)), _rows((_N, _D))],
        out_specs=[_rows((_N, 1)), _rows((_N, _CB)), _rows((_N, _CB))],
        out_shape=[jax.ShapeDtypeStruct((_N, 1), jnp.float32),
                   jax.ShapeDtypeStruct((_N, _CB), jnp.float32),
                   jax.ShapeDtypeStruct((_N, _CB), jnp.float32)],
    )(deg2, x)


def _tc2_body(s0, s1, u0, u1, dinv, w1, b1, o0, o1, o2, o3):
    dv = dinv[...]
    t0 = (s0[0] + s0[1] - u0[...]) * dv
    t1 = (s1[0] + s1[1] - u1[...]) * dv
    t = jnp.concatenate([t0, t1], axis=1)
    h = jnp.dot(t, w1[...], preferred_element_type=jnp.float32) + b1[...]
    h = jnp.maximum(h, 0.0) * dv
    o0[...] = h[:, 0 * _CB:1 * _CB]
    o1[...] = h[:, 1 * _CB:2 * _CB]
    o2[...] = h[:, 2 * _CB:3 * _CB]
    o3[...] = h[:, 3 * _CB:4 * _CB]


def _tc2(s0, s1, u0, u1, dinv, w1, b1):
    return pl.pallas_call(
        _tc2_body,
        grid=(_N // _BT,),
        in_specs=[_mid((_NC, _N, _CB)), _mid((_NC, _N, _CB)),
                  _rows((_N, _CB)), _rows((_N, _CB)), _rows((_N, 1)),
                  _whole((_D, _H)), _whole((1, _H))],
        out_specs=[_rows((_N, _CB))] * 4,
        out_shape=[jax.ShapeDtypeStruct((_N, _CB), jnp.float32)] * 4,
    )(s0, s1, u0, u1, dinv, w1, b1)


def _tc3_body(s0, s1, s2, s3, u0, u1, u2, u3, dinv, w2, b2, wc, o0, o1):
    dv = dinv[...]
    ss = (s0, s1, s2, s3)
    uu = (u0, u1, u2, u3)
    t = jnp.concatenate([(s[0] + s[1] - u[...]) * dv for s, u in zip(ss, uu)],
                        axis=1)
    h = jnp.dot(t, w2[...], preferred_element_type=jnp.float32) + b2[...]
    h = jnp.maximum(h, 0.0)
    cc = jnp.dot(h, wc[...], preferred_element_type=jnp.float32) * dv
    o0[...] = cc[:, :_CB]
    o1[...] = cc[:, _CB:]


def _tc3(s, u, dinv, w2, b2, wc):
    return pl.pallas_call(
        _tc3_body,
        grid=(_N // _BT,),
        in_specs=[_mid((_NC, _N, _CB))] * 4
                 + [_rows((_N, _CB))] * 4
                 + [_rows((_N, 1)), _whole((_H, _H)), _whole((1, _H)),
                    _whole((_H, 2 * _L))],
        out_specs=[_rows((_N, _CB))] * 2,
        out_shape=[jax.ShapeDtypeStruct((_N, _CB), jnp.float32)] * 2,
    )(*s, *u, dinv, w2, b2, wc)


def _tc4_body(s0, s1, u0, u1, dinv, bmu, blv, eps,
              wd1, bd1, wd2, bd2, wd3, bd3,
              mu_o, lv_o, z_o, rec_o, zacc):
    i = pl.program_id(0)
    dv = dinv[...]
    mu = (s0[0] + s0[1] - u0[...]) * dv + bmu[...]
    lv = (s1[0] + s1[1] - u1[...]) * dv + blv[...]
    std = jnp.exp(0.5 * lv)
    z = mu + eps[...] * std
    mu_o[...] = mu
    lv_o[...] = lv
    z_o[...] = z

    @pl.when(i == 0)
    def _():
        zacc[...] = jnp.zeros_like(zacc)

    zacc[...] += jnp.sum(z, axis=0, keepdims=True)

    @pl.when(i == pl.num_programs(0) - 1)
    def _():
        ge = zacc[...] * (1.0 / _N)
        d1 = jnp.dot(ge, wd1[...], preferred_element_type=jnp.float32) + bd1[...]
        d1 = jnp.maximum(d1, 0.0)
        d2 = jnp.dot(d1, wd2[...], preferred_element_type=jnp.float32) + bd2[...]
        d2 = jnp.maximum(d2, 0.0)
        o = jnp.dot(d2, wd3[...], preferred_element_type=jnp.float32) + bd3[...]
        rec_o[...] = 1.0 / (1.0 + jnp.exp(-o))


def _tc4(s, u, dinv, bmu, blv, eps, wd1, bd1, wd2, bd2, wd3, bd3):
    return pl.pallas_call(
        _tc4_body,
        grid=(_N // _BT,),
        in_specs=[_mid((_NC, _N, _CB))] * 2
                 + [_rows((_N, _CB))] * 2
                 + [_rows((_N, 1)), _whole((1, _L)), _whole((1, _L)),
                    _rows((_N, _L)),
                    _whole((_L, _H)), _whole((1, _H)),
                    _whole((_H, _H)), _whole((1, _H)),
                    _whole((_H, _D)), _whole((1, _D))],
        out_specs=[_rows((_N, _L)), _rows((_N, _L)), _rows((_N, _L)),
                   _whole((1, _D))],
        out_shape=[jax.ShapeDtypeStruct((_N, _L), jnp.float32),
                   jax.ShapeDtypeStruct((_N, _L), jnp.float32),
                   jax.ShapeDtypeStruct((_N, _L), jnp.float32),
                   jax.ShapeDtypeStruct((1, _D), jnp.float32)],
        scratch_shapes=[pltpu.VMEM((1, _L), jnp.float32)],
    )(*s, *u, dinv, bmu, blv, eps, wd1, bd1, wd2, bd2, wd3, bd3)


def kernel(x, edge_index, W1, b1, W2, b2, Wmu, bmu, Wlv, blv,
           Wd1, bd1, Wd2, bd2, Wd3, bd3):
    f32 = jnp.float32
    src = edge_index[0]
    dst = edge_index[1]
    npad = _EPAD - _E
    src_flat = jnp.concatenate([src, jnp.zeros((npad,), jnp.int32)])
    dst_flat = jnp.concatenate([dst, jnp.full((npad,), _N, jnp.int32)])
    src_p = src_flat.reshape(_NW, _CPW, _CH)
    dst_p = dst_flat.reshape(_NW, _CPW, _CH)
    dst_d = dst_flat.reshape(_NW, _CPWD, _CHD)

    ones = jnp.ones((_RPT, _CB), f32)
    deg2 = _make_deg()(dst_d, ones)

    prop2 = _make_prop(2)
    prop4 = _make_prop(4)
    dinv, u0, u1 = _tc1(deg2, x)
    s0, s1 = prop2(u0, u1, src_p, dst_p)
    v = _tc2(s0, s1, u0, u1, dinv, W1, b1.reshape(1, _H))
    t = prop4(*v, src_p, dst_p)
    wc = jnp.concatenate([Wmu, Wlv], axis=1)
    w0, w1 = _tc3(t, v, dinv, W2, b2.reshape(1, _H), wc)
    r = prop2(w0, w1, src_p, dst_p)
    eps = jax.random.normal(jax.random.key(42), (_N, _L), dtype=f32)
    mu, lv, z, rec = _tc4(r, (w0, w1), dinv,
                          bmu.reshape(1, _L), blv.reshape(1, _L), eps,
                          Wd1, bd1.reshape(1, _H), Wd2, bd2.reshape(1, _H),
                          Wd3, bd3.reshape(1, _D))
    return rec, mu, lv, z
